# baseline (speedup 1.0000x reference)
import functools

import jax
import jax.numpy as jnp
from jax import lax
from jax.experimental import pallas as pl
from jax.experimental.pallas import tpu as pltpu
from jax.experimental.pallas import tpu_sc as plsc

N = 10000
D = 256
K = 256
R = 8
E = 160000

NPAD = 10240
EPAD = 163840
HK = K // 2
RPT = NPAD // 16
SE = 1024
NSTRIP = EPAD // SE
CH = 128
MCAP = SE + CH + 16


def _table_body(x_ref, w_ref, g_ref, t_ref):
    xb = x_ref[...]
    h = jnp.dot(xb, w_ref[0], preferred_element_type=jnp.float32)
    gvec = g_ref[pl.program_id(1)]
    logit = jnp.sum(xb * gvec[None, :], axis=1, keepdims=True)
    h = h * jax.nn.sigmoid(logit)
    t_ref[0] = h[:, :HK]
    t_ref[1] = h[:, HK:]


def _make_table(x_pad, weight, gw):
    nb = NPAD // 1024
    return pl.pallas_call(
        _table_body,
        grid=(nb, R),
        in_specs=[
            pl.BlockSpec((1024, D), lambda n, r: (n, 0)),
            pl.BlockSpec((1, D, K), lambda n, r: (r, 0, 0)),
            pl.BlockSpec((R, D), lambda n, r: (0, 0)),
        ],
        out_specs=pl.BlockSpec((2, 1024, HK), lambda n, r: (0, r * nb + n, 0)),
        out_shape=jax.ShapeDtypeStruct((2, R * NPAD, HK), jnp.float32),
    )(x_pad, weight, gw)


def _sc_body(t_hbm, src_hbm, rel_hbm, dst_hbm, norm_hbm, out_hbm,
             sbuf, rbuf, dbuf, nbuf, meid, mg, mdl, mnr, buf, acc,
             sem_m, sem_g):
    c = lax.axis_index("c")
    s = lax.axis_index("s")
    lo = s * RPT
    zero16 = jnp.zeros((16,), jnp.int32)
    zero16f = jnp.zeros((16,), jnp.float32)
    iota16 = lax.iota(jnp.int32, 16)

    @plsc.parallel_loop(0, RPT * (HK // 16), unroll=8)
    def _fill_z(k):
        acc[k // (HK // 16), pl.ds((k % (HK // 16)) * 16, 16)] = zero16f

    coff = c * (R * NPAD)

    def build(q, b):
        for gg in range(CH // 16):
            e16 = meid[pl.ds(q * CH + gg * 16, 16)]
            g16 = (plsc.load_gather(rbuf, [e16]) * NPAD
                   + plsc.load_gather(sbuf, [e16]) + coff)
            sl = pl.ds(gg * 16, 16)
            mg[b, sl] = g16
            mdl[b, sl] = plsc.load_gather(dbuf, [e16]) - lo
            mnr[b, sl] = plsc.load_gather(nbuf, [e16])

    def accum(b):
        def edge(e, carry):
            dl = mdl[b, pl.ds(e, 16)][0]
            se = mnr[b, pl.ds(e, 16)][0]
            for v in range(HK // 16):
                cs = pl.ds(v * 16, 16)
                acc[dl, cs] = acc[dl, cs] + buf[b, e, cs] * se
            return carry
        lax.fori_loop(0, CH, edge, 0, unroll=2)

    def fire(b):
        pltpu.async_copy(t_hbm.at[mg.at[b]], buf.at[b], sem_g)

    def drain():
        pltpu.make_async_copy(t_hbm.at[pl.ds(0, CH)], buf.at[0],
                              sem_g).wait()

    def strip(t, carry):
        base = t * SE
        d1 = pltpu.async_copy(src_hbm.at[pl.ds(base, SE)],
                              sbuf.at[pl.ds(0, SE)], sem_m)
        d2 = pltpu.async_copy(rel_hbm.at[pl.ds(base, SE)],
                              rbuf.at[pl.ds(0, SE)], sem_m)
        d3 = pltpu.async_copy(dst_hbm.at[pl.ds(base, SE)],
                              dbuf.at[pl.ds(0, SE)], sem_m)
        d4 = pltpu.async_copy(norm_hbm.at[pl.ds(base, SE)],
                              nbuf.at[pl.ds(0, SE)], sem_m)
        d1.wait(); d2.wait(); d3.wait(); d4.wait()
        sbuf[pl.ds(SE, 16)] = zero16
        rbuf[pl.ds(SE, 16)] = zero16
        dbuf[pl.ds(SE, 16)] = zero16 + lo
        nbuf[pl.ds(SE, 16)] = zero16f

        @plsc.parallel_loop(0, SE // 16, unroll=4, carry=jnp.int32(0))
        def scan(g, m):
            dl = dbuf[pl.ds(g * 16, 16)] - lo
            ok = (dl >= 0) & (dl < RPT)
            cum = plsc.cumsum(jnp.where(ok, 1, 0))
            plsc.store_scatter(meid, [m + cum - 1], iota16 + g * 16,
                               mask=ok)
            return m + cum[15]
        m = scan

        for q in range(CH // 16):
            meid[pl.ds(m + q * 16, 16)] = zero16 + SE

        nch = jnp.maximum((m + CH - 1) // CH, 1)

        build(0, 0)
        fire(0)

        def chunk(q, carry2):
            b = q & 1
            build(q, b)
            fire(b)
            drain()
            accum(1 - b)
            return carry2
        lax.fori_loop(1, nch, chunk, 0)
        drain()
        accum((nch - 1) & 1)
        return carry
    lax.fori_loop(0, NSTRIP, strip, 0)

    pltpu.sync_copy(acc, out_hbm.at[c, pl.ds(lo, RPT)])


def _sc_aggregate(table, src_p, rel_p, dst_p, norm_p):
    mesh = plsc.VectorSubcoreMesh(core_axis_name="c", subcore_axis_name="s")
    f = pl.kernel(
        _sc_body,
        out_type=jax.ShapeDtypeStruct((2, NPAD, HK), jnp.float32),
        mesh=mesh,
        compiler_params=pltpu.CompilerParams(needs_layout_passes=False),
        scratch_types=[
            pltpu.VMEM((SE + 16,), jnp.int32),
            pltpu.VMEM((SE + 16,), jnp.int32),
            pltpu.VMEM((SE + 16,), jnp.int32),
            pltpu.VMEM((SE + 16,), jnp.float32),
            pltpu.VMEM((MCAP,), jnp.int32),
            pltpu.VMEM((2, CH), jnp.int32),
            pltpu.VMEM((2, CH + 16), jnp.int32),
            pltpu.VMEM((2, CH + 16), jnp.float32),
            pltpu.VMEM((2, CH, HK), jnp.float32),
            pltpu.VMEM((RPT, HK), jnp.float32),
            pltpu.SemaphoreType.DMA,
            pltpu.SemaphoreType.DMA,
        ],
    )
    return f(table, src_p, rel_p, dst_p, norm_p)


def _combine_body(p_ref, o_ref):
    o_ref[:, :HK] = jnp.maximum(p_ref[0], 0.0)
    o_ref[:, HK:] = jnp.maximum(p_ref[1], 0.0)


def _combine(parts):
    nb = NPAD // 1024
    return pl.pallas_call(
        _combine_body,
        grid=(nb,),
        in_specs=[pl.BlockSpec((2, 1024, HK), lambda i: (0, i, 0))],
        out_specs=pl.BlockSpec((1024, K), lambda i: (i, 0)),
        out_shape=jax.ShapeDtypeStruct((NPAD, K), jnp.float32),
    )(parts)


def kernel(x, edge_index, rel_type, norm, weight, gate_weight):
    x_pad = jnp.pad(x, ((0, NPAD - N), (0, 0)))
    pad = EPAD - E
    spread = (jnp.arange(pad, dtype=jnp.int32) * 16) % N
    src_p = jnp.concatenate([edge_index[0], spread])
    dst_p = jnp.concatenate([edge_index[1], spread])
    rel_p = jnp.pad(rel_type, (0, pad))
    norm_p = jnp.pad(norm, (0, pad))
    gw = gate_weight[:, :, 0]

    table = _make_table(x_pad, weight, gw).reshape(2 * R * NPAD, HK)
    parts = _sc_aggregate(table, src_p, rel_p, dst_p, norm_p)
    out = _combine(parts)
    return out[:N]


# speedup vs baseline: 3.5516x; 3.5516x over previous
import functools

import jax
import jax.numpy as jnp
from jax import lax
from jax.experimental import pallas as pl
from jax.experimental.pallas import tpu as pltpu
from jax.experimental.pallas import tpu_sc as plsc

N = 10000
D = 256
K = 256
R = 8
E = 160000

NPAD = 10240
EPAD = 163840
HK = K // 2
RPT = NPAD // 16
SE = 1024
NSTRIP = EPAD // SE
CH = 32
MCAP = SE + CH + 16


def _table_body(x_ref, w_ref, g_ref, t_ref):
    xb = x_ref[...]
    h = jnp.dot(xb, w_ref[0], preferred_element_type=jnp.float32)
    gvec = g_ref[pl.program_id(1)]
    logit = jnp.sum(xb * gvec[None, :], axis=1, keepdims=True)
    h = h * jax.nn.sigmoid(logit)
    t_ref[0] = h[:, :HK]
    t_ref[1] = h[:, HK:]


def _make_table(x_pad, weight, gw):
    nb = NPAD // 1024
    return pl.pallas_call(
        _table_body,
        grid=(nb, R),
        in_specs=[
            pl.BlockSpec((1024, D), lambda n, r: (n, 0)),
            pl.BlockSpec((1, D, K), lambda n, r: (r, 0, 0)),
            pl.BlockSpec((R, D), lambda n, r: (0, 0)),
        ],
        out_specs=pl.BlockSpec((2, 1024, HK), lambda n, r: (0, r * nb + n, 0)),
        out_shape=jax.ShapeDtypeStruct((2, R * NPAD, HK), jnp.float32),
    )(x_pad, weight, gw)


def _sc_body(t_hbm, src_hbm, rel_hbm, dst_hbm, norm_hbm, out_hbm,
             sbuf, rbuf, dbuf, nbuf, meid, mg, mdl, mnr, buf, acc,
             sem_m, sem_g):
    c = lax.axis_index("c")
    s = lax.axis_index("s")
    lo = s * RPT
    zero16 = jnp.zeros((16,), jnp.int32)
    zero16f = jnp.zeros((16,), jnp.float32)
    iota16 = lax.iota(jnp.int32, 16)

    @plsc.parallel_loop(0, RPT * (HK // 16), unroll=8)
    def _fill_z(k):
        acc[k // (HK // 16), pl.ds((k % (HK // 16)) * 16, 16)] = zero16f

    coff = c * (R * NPAD)

    def build(q, b):
        for gg in range(CH // 16):
            e16 = meid[pl.ds(q * CH + gg * 16, 16)]
            g16 = (plsc.load_gather(rbuf, [e16]) * NPAD
                   + plsc.load_gather(sbuf, [e16]) + coff)
            sl = pl.ds(gg * 16, 16)
            mg[b, sl] = g16
            mdl[b, sl] = plsc.load_gather(dbuf, [e16]) - lo
            mnr[b, sl] = plsc.load_gather(nbuf, [e16])

    def accum(b):
        def edge(e, carry):
            dl = mdl[b, pl.ds(e, 16)][0]
            se = mnr[b, pl.ds(e, 16)][0]
            for v in range(HK // 16):
                cs = pl.ds(v * 16, 16)
                acc[dl, cs] = acc[dl, cs] + buf[b, e, cs] * se
            return carry
        lax.fori_loop(0, CH, edge, 0, unroll=2)

    def fire(b):
        pltpu.async_copy(t_hbm.at[mg.at[b]], buf.at[b], sem_g)

    def drain():
        pltpu.make_async_copy(t_hbm.at[pl.ds(0, CH)], buf.at[0],
                              sem_g).wait()

    def strip(t, carry):
        base = t * SE
        d1 = pltpu.async_copy(src_hbm.at[pl.ds(base, SE)],
                              sbuf.at[pl.ds(0, SE)], sem_m)
        d2 = pltpu.async_copy(rel_hbm.at[pl.ds(base, SE)],
                              rbuf.at[pl.ds(0, SE)], sem_m)
        d3 = pltpu.async_copy(dst_hbm.at[pl.ds(base, SE)],
                              dbuf.at[pl.ds(0, SE)], sem_m)
        d4 = pltpu.async_copy(norm_hbm.at[pl.ds(base, SE)],
                              nbuf.at[pl.ds(0, SE)], sem_m)
        d1.wait(); d2.wait(); d3.wait(); d4.wait()
        sbuf[pl.ds(SE, 16)] = zero16
        rbuf[pl.ds(SE, 16)] = zero16
        dbuf[pl.ds(SE, 16)] = zero16 + lo
        nbuf[pl.ds(SE, 16)] = zero16f

        @plsc.parallel_loop(0, SE // 16, unroll=4, carry=jnp.int32(0))
        def scan(g, m):
            dl = dbuf[pl.ds(g * 16, 16)] - lo
            ok = (dl >= 0) & (dl < RPT)
            cum = plsc.cumsum(jnp.where(ok, 1, 0))
            plsc.store_scatter(meid, [m + cum - 1], iota16 + g * 16,
                               mask=ok)
            return m + cum[15]
        m = scan

        for q in range(CH // 16):
            meid[pl.ds(m + q * 16, 16)] = zero16 + SE

        nch = jnp.maximum((m + CH - 1) // CH, 1)

        build(0, 0)
        fire(0)

        def chunk(q, carry2):
            b = q & 1
            build(q, b)
            fire(b)
            drain()
            accum(1 - b)
            return carry2
        lax.fori_loop(1, nch, chunk, 0)
        drain()
        accum((nch - 1) & 1)
        return carry
    lax.fori_loop(0, NSTRIP, strip, 0)

    pltpu.sync_copy(acc, out_hbm.at[c, pl.ds(lo, RPT)])


def _sc_aggregate(table, src_p, rel_p, dst_p, norm_p):
    mesh = plsc.VectorSubcoreMesh(core_axis_name="c", subcore_axis_name="s")
    f = pl.kernel(
        _sc_body,
        out_type=jax.ShapeDtypeStruct((2, NPAD, HK), jnp.float32),
        mesh=mesh,
        compiler_params=pltpu.CompilerParams(needs_layout_passes=False),
        scratch_types=[
            pltpu.VMEM((SE + 16,), jnp.int32),
            pltpu.VMEM((SE + 16,), jnp.int32),
            pltpu.VMEM((SE + 16,), jnp.int32),
            pltpu.VMEM((SE + 16,), jnp.float32),
            pltpu.VMEM((MCAP,), jnp.int32),
            pltpu.VMEM((2, CH), jnp.int32),
            pltpu.VMEM((2, CH + 16), jnp.int32),
            pltpu.VMEM((2, CH + 16), jnp.float32),
            pltpu.VMEM((2, CH, HK), jnp.float32),
            pltpu.VMEM((RPT, HK), jnp.float32),
            pltpu.SemaphoreType.DMA,
            pltpu.SemaphoreType.DMA,
        ],
    )
    return f(table, src_p, rel_p, dst_p, norm_p)


def _combine_body(p_ref, o_ref):
    o_ref[:, :HK] = jnp.maximum(p_ref[0], 0.0)
    o_ref[:, HK:] = jnp.maximum(p_ref[1], 0.0)


def _combine(parts):
    nb = NPAD // 1024
    return pl.pallas_call(
        _combine_body,
        grid=(nb,),
        in_specs=[pl.BlockSpec((2, 1024, HK), lambda i: (0, i, 0))],
        out_specs=pl.BlockSpec((1024, K), lambda i: (i, 0)),
        out_shape=jax.ShapeDtypeStruct((NPAD, K), jnp.float32),
    )(parts)


def kernel(x, edge_index, rel_type, norm, weight, gate_weight):
    x_pad = jnp.pad(x, ((0, NPAD - N), (0, 0)))
    pad = EPAD - E
    spread = (jnp.arange(pad, dtype=jnp.int32) * 16) % N
    src_p = jnp.concatenate([edge_index[0], spread])
    dst_p = jnp.concatenate([edge_index[1], spread])
    rel_p = jnp.pad(rel_type, (0, pad))
    norm_p = jnp.pad(norm, (0, pad))
    gw = gate_weight[:, :, 0]

    table = _make_table(x_pad, weight, gw).reshape(2 * R * NPAD, HK)
    parts = _sc_aggregate(table, src_p, rel_p, dst_p, norm_p)
    out = _combine(parts)
    return out[:N]


# speedup vs baseline: 5.5179x; 1.5536x over previous
import functools

import jax
import jax.numpy as jnp
from jax import lax
from jax.experimental import pallas as pl
from jax.experimental.pallas import tpu as pltpu
from jax.experimental.pallas import tpu_sc as plsc

N = 10000
D = 256
K = 256
R = 8
E = 160000

NPAD = 10240
EPAD = 163840
HK = K // 2
RPT = NPAD // 16
SE = 1024
NSTRIP = EPAD // SE
CH = 16
MCAP = SE + CH + 16


def _table_body(x_ref, w_ref, g_ref, t_ref):
    xb = x_ref[...]
    h = jnp.dot(xb, w_ref[0], preferred_element_type=jnp.float32)
    gvec = g_ref[pl.program_id(1)]
    logit = jnp.sum(xb * gvec[None, :], axis=1, keepdims=True)
    h = h * jax.nn.sigmoid(logit)
    t_ref[0] = h[:, :HK]
    t_ref[1] = h[:, HK:]


def _make_table(x_pad, weight, gw):
    nb = NPAD // 1024
    return pl.pallas_call(
        _table_body,
        grid=(nb, R),
        in_specs=[
            pl.BlockSpec((1024, D), lambda n, r: (n, 0)),
            pl.BlockSpec((1, D, K), lambda n, r: (r, 0, 0)),
            pl.BlockSpec((R, D), lambda n, r: (0, 0)),
        ],
        out_specs=pl.BlockSpec((2, 1024, HK), lambda n, r: (0, r * nb + n, 0)),
        out_shape=jax.ShapeDtypeStruct((2, R * NPAD, HK), jnp.float32),
    )(x_pad, weight, gw)


def _sc_body(t_hbm, src_hbm, rel_hbm, dst_hbm, norm_hbm, out_hbm,
             sbuf, rbuf, dbuf, nbuf, meid, mg, mdl, mnr, buf, acc,
             sem_m, sem_g):
    c = lax.axis_index("c")
    s = lax.axis_index("s")
    lo = s * RPT
    zero16 = jnp.zeros((16,), jnp.int32)
    zero16f = jnp.zeros((16,), jnp.float32)
    iota16 = lax.iota(jnp.int32, 16)

    @plsc.parallel_loop(0, RPT * (HK // 16), unroll=8)
    def _fill_z(k):
        acc[k // (HK // 16), pl.ds((k % (HK // 16)) * 16, 16)] = zero16f

    coff = c * (R * NPAD)

    def build(q, b):
        for gg in range(CH // 16):
            e16 = meid[pl.ds(q * CH + gg * 16, 16)]
            g16 = (plsc.load_gather(rbuf, [e16]) * NPAD
                   + plsc.load_gather(sbuf, [e16]) + coff)
            sl = pl.ds(gg * 16, 16)
            mg[b, sl] = g16
            mdl[b, sl] = plsc.load_gather(dbuf, [e16]) - lo
            mnr[b, sl] = plsc.load_gather(nbuf, [e16])

    def accum(b):
        def edge(e, carry):
            dl = mdl[b, pl.ds(e, 16)][0]
            se = mnr[b, pl.ds(e, 16)][0]
            for v in range(HK // 16):
                cs = pl.ds(v * 16, 16)
                acc[dl, cs] = acc[dl, cs] + buf[b, e, cs] * se
            return carry
        lax.fori_loop(0, CH, edge, 0, unroll=2)

    def fire(b):
        pltpu.async_copy(t_hbm.at[mg.at[b]], buf.at[b], sem_g)

    def drain():
        pltpu.make_async_copy(t_hbm.at[pl.ds(0, CH)], buf.at[0],
                              sem_g).wait()

    def strip(t, carry):
        base = t * SE
        d1 = pltpu.async_copy(src_hbm.at[pl.ds(base, SE)],
                              sbuf.at[pl.ds(0, SE)], sem_m)
        d2 = pltpu.async_copy(rel_hbm.at[pl.ds(base, SE)],
                              rbuf.at[pl.ds(0, SE)], sem_m)
        d3 = pltpu.async_copy(dst_hbm.at[pl.ds(base, SE)],
                              dbuf.at[pl.ds(0, SE)], sem_m)
        d4 = pltpu.async_copy(norm_hbm.at[pl.ds(base, SE)],
                              nbuf.at[pl.ds(0, SE)], sem_m)
        d1.wait(); d2.wait(); d3.wait(); d4.wait()
        sbuf[pl.ds(SE, 16)] = zero16
        rbuf[pl.ds(SE, 16)] = zero16
        dbuf[pl.ds(SE, 16)] = zero16 + lo
        nbuf[pl.ds(SE, 16)] = zero16f

        @plsc.parallel_loop(0, SE // 16, unroll=4, carry=jnp.int32(0))
        def scan(g, m):
            dl = dbuf[pl.ds(g * 16, 16)] - lo
            ok = (dl >= 0) & (dl < RPT)
            cum = plsc.cumsum(jnp.where(ok, 1, 0))
            plsc.store_scatter(meid, [m + cum - 1], iota16 + g * 16,
                               mask=ok)
            return m + cum[15]
        m = scan

        for q in range(CH // 16):
            meid[pl.ds(m + q * 16, 16)] = zero16 + SE

        nch = jnp.maximum((m + CH - 1) // CH, 1)

        build(0, 0)
        fire(0)

        def chunk(q, carry2):
            b = q & 1
            build(q, b)
            fire(b)
            drain()
            accum(1 - b)
            return carry2
        lax.fori_loop(1, nch, chunk, 0)
        drain()
        accum((nch - 1) & 1)
        return carry
    lax.fori_loop(0, NSTRIP, strip, 0)

    pltpu.sync_copy(acc, out_hbm.at[c, pl.ds(lo, RPT)])


def _sc_aggregate(table, src_p, rel_p, dst_p, norm_p):
    mesh = plsc.VectorSubcoreMesh(core_axis_name="c", subcore_axis_name="s")
    f = pl.kernel(
        _sc_body,
        out_type=jax.ShapeDtypeStruct((2, NPAD, HK), jnp.float32),
        mesh=mesh,
        compiler_params=pltpu.CompilerParams(needs_layout_passes=False),
        scratch_types=[
            pltpu.VMEM((SE + 16,), jnp.int32),
            pltpu.VMEM((SE + 16,), jnp.int32),
            pltpu.VMEM((SE + 16,), jnp.int32),
            pltpu.VMEM((SE + 16,), jnp.float32),
            pltpu.VMEM((MCAP,), jnp.int32),
            pltpu.VMEM((2, CH), jnp.int32),
            pltpu.VMEM((2, CH + 16), jnp.int32),
            pltpu.VMEM((2, CH + 16), jnp.float32),
            pltpu.VMEM((2, CH, HK), jnp.float32),
            pltpu.VMEM((RPT, HK), jnp.float32),
            pltpu.SemaphoreType.DMA,
            pltpu.SemaphoreType.DMA,
        ],
    )
    return f(table, src_p, rel_p, dst_p, norm_p)


def _combine_body(p_ref, o_ref):
    o_ref[:, :HK] = jnp.maximum(p_ref[0], 0.0)
    o_ref[:, HK:] = jnp.maximum(p_ref[1], 0.0)


def _combine(parts):
    nb = NPAD // 1024
    return pl.pallas_call(
        _combine_body,
        grid=(nb,),
        in_specs=[pl.BlockSpec((2, 1024, HK), lambda i: (0, i, 0))],
        out_specs=pl.BlockSpec((1024, K), lambda i: (i, 0)),
        out_shape=jax.ShapeDtypeStruct((NPAD, K), jnp.float32),
    )(parts)


def kernel(x, edge_index, rel_type, norm, weight, gate_weight):
    x_pad = jnp.pad(x, ((0, NPAD - N), (0, 0)))
    pad = EPAD - E
    spread = (jnp.arange(pad, dtype=jnp.int32) * 16) % N
    src_p = jnp.concatenate([edge_index[0], spread])
    dst_p = jnp.concatenate([edge_index[1], spread])
    rel_p = jnp.pad(rel_type, (0, pad))
    norm_p = jnp.pad(norm, (0, pad))
    gw = gate_weight[:, :, 0]

    table = _make_table(x_pad, weight, gw).reshape(2 * R * NPAD, HK)
    parts = _sc_aggregate(table, src_p, rel_p, dst_p, norm_p)
    out = _combine(parts)
    return out[:N]


# speedup vs baseline: 5.8430x; 1.0589x over previous
import functools

import jax
import jax.numpy as jnp
from jax import lax
from jax.experimental import pallas as pl
from jax.experimental.pallas import tpu as pltpu
from jax.experimental.pallas import tpu_sc as plsc

N = 10000
D = 256
K = 256
R = 8
E = 160000

NPAD = 10240
EPAD = 163840
HK = K // 2
RPT = NPAD // 16
SE = 1024
NSTRIP = EPAD // SE
CH = 16
NB = 8
MCAP = SE + CH + 16


def _table_body(x_ref, w_ref, g_ref, t_ref):
    xb = x_ref[...]
    h = jnp.dot(xb, w_ref[0], preferred_element_type=jnp.float32)
    gvec = g_ref[pl.program_id(1)]
    logit = jnp.sum(xb * gvec[None, :], axis=1, keepdims=True)
    h = h * jax.nn.sigmoid(logit)
    t_ref[0] = h[:, :HK]
    t_ref[1] = h[:, HK:]


def _make_table(x_pad, weight, gw):
    nb = NPAD // 1024
    return pl.pallas_call(
        _table_body,
        grid=(nb, R),
        in_specs=[
            pl.BlockSpec((1024, D), lambda n, r: (n, 0)),
            pl.BlockSpec((1, D, K), lambda n, r: (r, 0, 0)),
            pl.BlockSpec((R, D), lambda n, r: (0, 0)),
        ],
        out_specs=pl.BlockSpec((2, 1024, HK), lambda n, r: (0, r * nb + n, 0)),
        out_shape=jax.ShapeDtypeStruct((2, R * NPAD, HK), jnp.float32),
    )(x_pad, weight, gw)


def _sc_body(t_hbm, src_hbm, rel_hbm, dst_hbm, norm_hbm, out_hbm,
             sbuf, rbuf, dbuf, nbuf, meid, mg, mdl, mnr, buf, acc,
             sem_m, sem_g):
    c = lax.axis_index("c")
    s = lax.axis_index("s")
    lo = s * RPT
    zero16 = jnp.zeros((16,), jnp.int32)
    zero16f = jnp.zeros((16,), jnp.float32)
    iota16 = lax.iota(jnp.int32, 16)

    @plsc.parallel_loop(0, RPT * (HK // 16), unroll=8)
    def _fill_z(k):
        acc[k // (HK // 16), pl.ds((k % (HK // 16)) * 16, 16)] = zero16f

    coff = c * (R * NPAD)

    def build(q, b):
        for gg in range(CH // 16):
            e16 = meid[pl.ds(q * CH + gg * 16, 16)]
            g16 = (plsc.load_gather(rbuf, [e16]) * NPAD
                   + plsc.load_gather(sbuf, [e16]) + coff)
            sl = pl.ds(gg * 16, 16)
            mg[b, sl] = g16
            mdl[b, sl] = plsc.load_gather(dbuf, [e16]) - lo
            mnr[b, sl] = plsc.load_gather(nbuf, [e16])

    def accum(b):
        def edge(e, carry):
            dl = mdl[b, pl.ds(e, 16)][0]
            se = mnr[b, pl.ds(e, 16)][0]
            for v in range(HK // 16):
                cs = pl.ds(v * 16, 16)
                acc[dl, cs] = acc[dl, cs] + buf[b, e, cs] * se
            return carry
        lax.fori_loop(0, CH, edge, 0, unroll=2)

    def fire(b):
        pltpu.async_copy(t_hbm.at[mg.at[b]], buf.at[b], sem_g)

    def drain():
        pltpu.make_async_copy(t_hbm.at[pl.ds(0, CH)], buf.at[0],
                              sem_g).wait()

    def strip(t, carry):
        base = t * SE
        d1 = pltpu.async_copy(src_hbm.at[pl.ds(base, SE)],
                              sbuf.at[pl.ds(0, SE)], sem_m)
        d2 = pltpu.async_copy(rel_hbm.at[pl.ds(base, SE)],
                              rbuf.at[pl.ds(0, SE)], sem_m)
        d3 = pltpu.async_copy(dst_hbm.at[pl.ds(base, SE)],
                              dbuf.at[pl.ds(0, SE)], sem_m)
        d4 = pltpu.async_copy(norm_hbm.at[pl.ds(base, SE)],
                              nbuf.at[pl.ds(0, SE)], sem_m)
        d1.wait(); d2.wait(); d3.wait(); d4.wait()
        sbuf[pl.ds(SE, 16)] = zero16
        rbuf[pl.ds(SE, 16)] = zero16
        dbuf[pl.ds(SE, 16)] = zero16 + lo
        nbuf[pl.ds(SE, 16)] = zero16f

        @plsc.parallel_loop(0, SE // 16, unroll=4, carry=jnp.int32(0))
        def scan(g, m):
            dl = dbuf[pl.ds(g * 16, 16)] - lo
            ok = (dl >= 0) & (dl < RPT)
            cum = plsc.cumsum(jnp.where(ok, 1, 0))
            plsc.store_scatter(meid, [m + cum - 1], iota16 + g * 16,
                               mask=ok)
            return m + cum[15]
        m = scan

        for q in range(CH // 16):
            meid[pl.ds(m + q * 16, 16)] = zero16 + SE

        nch = jnp.maximum((m + CH - 1) // CH, 1)

        def chunk(q, carry2):
            @pl.when(q >= NB)
            def _tail():
                drain()
                accum((q - NB) % NB)

            @pl.when(q < nch)
            def _head():
                b = q % NB
                build(q, b)
                fire(b)
            return carry2
        lax.fori_loop(0, nch + NB, chunk, 0)
        return carry
    lax.fori_loop(0, NSTRIP, strip, 0)

    pltpu.sync_copy(acc, out_hbm.at[c, pl.ds(lo, RPT)])


def _sc_aggregate(table, src_p, rel_p, dst_p, norm_p):
    mesh = plsc.VectorSubcoreMesh(core_axis_name="c", subcore_axis_name="s")
    f = pl.kernel(
        _sc_body,
        out_type=jax.ShapeDtypeStruct((2, NPAD, HK), jnp.float32),
        mesh=mesh,
        compiler_params=pltpu.CompilerParams(needs_layout_passes=False),
        scratch_types=[
            pltpu.VMEM((SE + 16,), jnp.int32),
            pltpu.VMEM((SE + 16,), jnp.int32),
            pltpu.VMEM((SE + 16,), jnp.int32),
            pltpu.VMEM((SE + 16,), jnp.float32),
            pltpu.VMEM((MCAP,), jnp.int32),
            pltpu.VMEM((NB, CH), jnp.int32),
            pltpu.VMEM((NB, CH + 16), jnp.int32),
            pltpu.VMEM((NB, CH + 16), jnp.float32),
            pltpu.VMEM((NB, CH, HK), jnp.float32),
            pltpu.VMEM((RPT, HK), jnp.float32),
            pltpu.SemaphoreType.DMA,
            pltpu.SemaphoreType.DMA,
        ],
    )
    return f(table, src_p, rel_p, dst_p, norm_p)


def _combine_body(p_ref, o_ref):
    o_ref[:, :HK] = jnp.maximum(p_ref[0], 0.0)
    o_ref[:, HK:] = jnp.maximum(p_ref[1], 0.0)


def _combine(parts):
    nb = NPAD // 1024
    return pl.pallas_call(
        _combine_body,
        grid=(nb,),
        in_specs=[pl.BlockSpec((2, 1024, HK), lambda i: (0, i, 0))],
        out_specs=pl.BlockSpec((1024, K), lambda i: (i, 0)),
        out_shape=jax.ShapeDtypeStruct((NPAD, K), jnp.float32),
    )(parts)


def kernel(x, edge_index, rel_type, norm, weight, gate_weight):
    x_pad = jnp.pad(x, ((0, NPAD - N), (0, 0)))
    pad = EPAD - E
    spread = (jnp.arange(pad, dtype=jnp.int32) * 16) % N
    src_p = jnp.concatenate([edge_index[0], spread])
    dst_p = jnp.concatenate([edge_index[1], spread])
    rel_p = jnp.pad(rel_type, (0, pad))
    norm_p = jnp.pad(norm, (0, pad))
    gw = gate_weight[:, :, 0]

    table = _make_table(x_pad, weight, gw).reshape(2 * R * NPAD, HK)
    parts = _sc_aggregate(table, src_p, rel_p, dst_p, norm_p)
    out = _combine(parts)
    return out[:N]
